# Initial kernel scaffold; baseline (speedup 1.0000x reference)
import jax
import jax.numpy as jnp
from jax.experimental import pallas as pl


def kernel(x, edge_index, batch, w):
    raise NotImplementedError("write your pallas kernel here")


# speedup vs baseline: 10.1893x; 10.1893x over previous
import jax
import jax.numpy as jnp
from jax.experimental import pallas as pl
from jax.experimental.pallas import tpu as pltpu

_N = 10000
_NPAD = 10240
_G = 64
_D = 256
_NBINS = 256


def _body(x_ref, brow_ref, w_ref, out_ref):
    x = x_ref[...]
    brow = brow_ref[...]
    w_row = w_ref[...]

    f32 = jnp.float32
    dg = jax.lax.dot_general
    wn = jax.lax.rsqrt(jnp.sum(w_row * w_row))
    score = dg(w_row, x, (((1,), (1,)), ((), ())),
               preferred_element_type=f32) * wn

    sc = jnp.where(score == 0.0, 0.0, score)
    sbits = jax.lax.bitcast_convert_type(sc, jnp.int32)
    key_asc = jnp.where(sbits >= 0, sbits, sbits ^ jnp.int32(0x7FFFFFFF))
    fk = -key_asc
    idx = jax.lax.broadcasted_iota(jnp.int32, (1, _NPAD), 1)

    digits = [
        ((fk >> 24) & 255) ^ 128,
        (fk >> 16) & 255,
        (fk >> 8) & 255,
        fk & 255,
        (idx >> 8) & 255,
        idx & 255,
    ]

    gids_col = jax.lax.broadcasted_iota(jnp.int32, (_G, 1), 0)
    ohT = jnp.where(gids_col == brow, 1.0, 0.0).astype(f32)

    ones_row = jnp.ones((1, _NPAD), f32)
    counts = dg(ones_row, ohT, (((1,), (1,)), ((), ())),
                preferred_element_type=f32)
    k = jnp.ceil(0.5 * counts)

    bins_col = jax.lax.broadcasted_iota(jnp.int32, (_NBINS, 1), 0)
    bins_row = jax.lax.broadcasted_iota(jnp.int32, (1, _NBINS), 1)
    binsf_col = bins_col.astype(f32)
    l2 = jnp.where(bins_row <= bins_col, 1.0, 0.0).astype(f32)

    act = brow < _G
    lt = jnp.zeros((1, _NPAD), jnp.bool_)
    r = k

    for d in digits:
        df = d.astype(f32)
        mT = jnp.where(act & (d == bins_col), 1.0, 0.0).astype(f32)
        histT = dg(mT, ohT, (((1,), (1,)), ((), ())),
                   preferred_element_type=f32)
        cumT = dg(l2, histT, (((1,), (0,)), ((), ())),
                  preferred_element_type=f32)
        ge = cumT >= r
        bsel = jnp.min(jnp.where(ge, binsf_col, 256.0), axis=0,
                       keepdims=True)
        bo = binsf_col == bsel
        hist_at = jnp.sum(jnp.where(bo, histT, 0.0), axis=0, keepdims=True)
        cum_at = jnp.sum(jnp.where(bo, cumT, 0.0), axis=0, keepdims=True)
        r = r - (cum_at - hist_at)
        dsel_at = dg(bsel, ohT, (((1,), (0,)), ((), ())),
                     preferred_element_type=f32)
        lt = lt | (act & (df < dsel_at))
        act = act & (df == dsel_at)

    sel = lt | act
    gate = jnp.tanh(score)
    wsel = jnp.where(sel, gate, 0.0)
    ohT_w = ohT * wsel
    pooled = dg(ohT_w, x, (((1,), (0,)), ((), ())),
                preferred_element_type=f32)
    inv = 1.0 / jnp.maximum(k, 1.0)
    gids_row = jax.lax.broadcasted_iota(jnp.int32, (1, _G), 1)
    eye = jnp.where(gids_col == gids_row, 1.0, 0.0).astype(f32)
    inv_col = dg(eye, inv, (((1,), (1,)), ((), ())),
                 preferred_element_type=f32)
    out_ref[...] = pooled * inv_col


def kernel(x, edge_index, batch, w):
    del edge_index
    xp = jnp.zeros((_NPAD, _D), jnp.float32).at[:_N].set(x)
    bp = jnp.full((_NPAD,), _G, jnp.int32).at[:_N].set(batch)
    out = pl.pallas_call(
        _body,
        out_shape=jax.ShapeDtypeStruct((_G, _D), jnp.float32),
    )(xp, bp.reshape(1, _NPAD), w.reshape(1, _D))
    return out
